# Initial kernel scaffold; baseline (speedup 1.0000x reference)
import jax
import jax.numpy as jnp
from jax.experimental import pallas as pl


def kernel(x, edge_index, edge_type, bases0, comp0, root0, bias0, bases1, comp1, root1, bias1, bases2, comp2, root2, bias2):
    raise NotImplementedError("write your pallas kernel here")


# speedup vs baseline: 22.8288x; 22.8288x over previous
import functools

import jax
import jax.numpy as jnp
from jax import lax
from jax.experimental import pallas as pl
from jax.experimental.pallas import tpu as pltpu
from jax.experimental.pallas import tpu_sc as plsc

NC = 2
NS = 16
NW = NC * NS
CH = 128


def _make_sc_scatter(N, D, ACC, n_chunks, with_cnt):
    stripe = ACC // NS
    mesh = plsc.VectorSubcoreMesh(core_axis_name="c", subcore_axis_name="s")
    out_type = [jax.ShapeDtypeStruct((NC, ACC, D), jnp.float32)]
    if with_cnt:
        out_type.append(jax.ShapeDtypeStruct((NC, ACC), jnp.float32))
    scratch = [
        pltpu.VMEM((n_chunks, CH), jnp.int32),
        pltpu.VMEM((n_chunks, CH), jnp.int32),
        pltpu.VMEM((n_chunks, CH), jnp.int32),
        pltpu.VMEM((CH,), jnp.int32),
        pltpu.VMEM((CH, D), jnp.float32),
        pltpu.VMEM((CH,), jnp.float32),
        pltpu.VMEM((stripe,), jnp.float32),
        pltpu.VMEM_SHARED((ACC, D), jnp.float32),
        pltpu.VMEM_SHARED((ACC,), jnp.float32),
        pltpu.SemaphoreType.DMA,
    ]

    def body(src_hbm, et_hbm, dst_hbm, hx_hbm, *rest):
        if with_cnt:
            out_hbm, cnt_hbm = rest[0], rest[1]
            rest = rest[2:]
        else:
            out_hbm = rest[0]
            cnt_hbm = None
            rest = rest[1:]
        (src2, et2, dst2, row_v, msg, ones_v, cz_v,
         acc_sh, cnt_sh, sem) = rest

        c = lax.axis_index("c")
        s = lax.axis_index("s")
        wid = s * NC + c

        pltpu.sync_copy(src_hbm.at[wid], src2)
        pltpu.sync_copy(et_hbm.at[wid], et2)
        pltpu.sync_copy(dst_hbm.at[wid], dst2)

        zer16 = jnp.zeros((16,), jnp.float32)

        def zrow(i, _):
            for j in range(D // 16):
                msg[i, pl.ds(j * 16, 16)] = zer16
            return 0
        lax.fori_loop(0, CH, zrow, 0)

        for j in range(CH // 16):
            ones_v[pl.ds(j * 16, 16)] = jnp.ones((16,), jnp.float32)

        def zc(i, _):
            cz_v[pl.ds(i * 16, 16)] = zer16
            return 0
        lax.fori_loop(0, stripe // 16, zc, 0)

        for q in range(stripe // CH):
            pltpu.sync_copy(msg, acc_sh.at[pl.ds(s * stripe + q * CH, CH), :])
        pltpu.sync_copy(cz_v, cnt_sh.at[pl.ds(s * stripe, stripe)])
        plsc.subcore_barrier()

        def step(k, _):
            for j in range(CH // 16):
                sl = pl.ds(j * 16, 16)
                row_v[sl] = et2[k, sl] * N + src2[k, sl]
            pltpu.async_copy(hx_hbm.at[row_v], msg, sem).wait()
            pltpu.sync_copy(msg, acc_sh.at[dst2.at[k]], add=True)
            if with_cnt:
                pltpu.sync_copy(ones_v, cnt_sh.at[dst2.at[k]], add=True)
            return 0
        lax.fori_loop(0, n_chunks, step, 0)
        plsc.subcore_barrier()

        pltpu.sync_copy(acc_sh.at[pl.ds(s * stripe, stripe), :],
                        out_hbm.at[c, pl.ds(s * stripe, stripe), :])
        if with_cnt:
            pltpu.sync_copy(cnt_sh.at[pl.ds(s * stripe, stripe)],
                            cnt_hbm.at[c, pl.ds(s * stripe, stripe)])

    return pl.kernel(body, mesh=mesh, out_type=out_type,
                     scratch_types=scratch)


def _expand_body(h_ref, bases_ref, comp_ref, root_ref, bias_ref,
                 hx_ref, rt_ref):
    h = h_ref[...]
    W = jnp.einsum('rb,bio->rio', comp_ref[...], bases_ref[...],
                   preferred_element_type=jnp.float32)
    hx_ref[...] = jnp.einsum('ni,rio->rno', h, W,
                             preferred_element_type=jnp.float32)
    rt_ref[...] = (jnp.dot(h, root_ref[...],
                           preferred_element_type=jnp.float32)
                   + bias_ref[...])


def _make_expand(N, D, R, B, nb):
    grid = (N // nb,)
    return pl.pallas_call(
        _expand_body,
        grid=grid,
        in_specs=[
            pl.BlockSpec((nb, D), lambda i: (i, 0)),
            pl.BlockSpec((B, D, D), lambda i: (0, 0, 0)),
            pl.BlockSpec((R, B), lambda i: (0, 0)),
            pl.BlockSpec((D, D), lambda i: (0, 0)),
            pl.BlockSpec((1, D), lambda i: (0, 0)),
        ],
        out_specs=[
            pl.BlockSpec((R, nb, D), lambda i: (0, i, 0)),
            pl.BlockSpec((nb, D), lambda i: (i, 0)),
        ],
        out_shape=[
            jax.ShapeDtypeStruct((R, N, D), jnp.float32),
            jax.ShapeDtypeStruct((N, D), jnp.float32),
        ],
    )


def _combine_body(relu, p_ref, cnt_ref, rt_ref, o_ref):
    sm = p_ref[0] + p_ref[1]
    cn = cnt_ref[0] + cnt_ref[1]
    o = sm / jnp.maximum(cn, 1.0) + rt_ref[...]
    if relu:
        o = jnp.maximum(o, 0.0)
    o_ref[...] = o


def _make_combine(N, D, ACC, nb, relu):
    grid = (N // nb,)
    return pl.pallas_call(
        functools.partial(_combine_body, relu),
        grid=grid,
        in_specs=[
            pl.BlockSpec((NC, nb, D), lambda i: (0, i, 0)),
            pl.BlockSpec((NC, nb, 1), lambda i: (0, i, 0)),
            pl.BlockSpec((nb, D), lambda i: (i, 0)),
        ],
        out_specs=pl.BlockSpec((nb, D), lambda i: (i, 0)),
        out_shape=jax.ShapeDtypeStruct((N, D), jnp.float32),
    )


def kernel(x, edge_index, edge_type,
           bases0, comp0, root0, bias0,
           bases1, comp1, root1, bias1,
           bases2, comp2, root2, bias2):
    N, D = x.shape
    E = edge_type.shape[0]
    R, B = comp0.shape

    epw = -(-E // (NW * CH)) * CH
    n_chunks = epw // CH
    EP = epw * NW
    padn = EP - E
    ACC = -(-(N + CH) // (NS * 8)) * (NS * 8)
    while ACC % NS or (ACC // NS) % CH:
        ACC += NS * 8
    stripe = ACC // NS
    assert stripe % CH == 0 and ACC > N

    src = edge_index[0]
    dst = edge_index[1]
    ar = jnp.arange(padn, dtype=jnp.int32)
    src_p = jnp.concatenate([src, ar % N]).reshape(NW, n_chunks, CH)
    et_p = jnp.concatenate(
        [edge_type, jnp.zeros((padn,), jnp.int32)]).reshape(NW, n_chunks, CH)
    dst_p = jnp.concatenate(
        [dst, N + (ar % (ACC - N))]).reshape(NW, n_chunks, CH)

    expand = _make_expand(N, D, R, B, nb=1000)
    sc0 = _make_sc_scatter(N, D, ACC, n_chunks, with_cnt=True)
    sc1 = _make_sc_scatter(N, D, ACC, n_chunks, with_cnt=False)

    params = [(bases0, comp0, root0, bias0),
              (bases1, comp1, root1, bias1),
              (bases2, comp2, root2, bias2)]

    h = x
    cnt3 = None
    for li, (bases, comp, root, bias) in enumerate(params):
        hx, rt = expand(h, bases, comp, root, bias.reshape(1, D))
        hx_flat = hx.reshape(R * N, D)
        if li == 0:
            parts, cnt = sc0(src_p, et_p, dst_p, hx_flat)
            cnt3 = cnt.reshape(NC, ACC, 1)
        else:
            (parts,) = sc1(src_p, et_p, dst_p, hx_flat)
        combine = _make_combine(N, D, ACC, nb=1000, relu=(li < 2))
        h = combine(parts, cnt3, rt)
    return h


# speedup vs baseline: 30.9607x; 1.3562x over previous
import functools

import jax
import jax.numpy as jnp
from jax import lax
from jax.experimental import pallas as pl
from jax.experimental.pallas import tpu as pltpu
from jax.experimental.pallas import tpu_sc as plsc

NC = 2
NS = 16
NW = NC * NS
CH = 128


def _make_sc_scatter(N, D, ACC, E, epw, n_chunks, with_cnt):
    stripe = ACC // NS
    mesh = plsc.VectorSubcoreMesh(core_axis_name="c", subcore_axis_name="s")
    out_type = [jax.ShapeDtypeStruct((NC, ACC, D), jnp.float32)]
    if with_cnt:
        out_type.append(jax.ShapeDtypeStruct((NC, ACC), jnp.float32))
    scratch = [
        pltpu.VMEM((n_chunks, CH), jnp.int32),
        pltpu.VMEM((CH,), jnp.int32),
        pltpu.VMEM((CH,), jnp.int32),
        pltpu.VMEM((CH, D), jnp.float32),
        pltpu.VMEM((CH, D), jnp.float32),
        pltpu.VMEM((CH,), jnp.float32),
        pltpu.VMEM((stripe,), jnp.float32),
        pltpu.VMEM_SHARED((ACC, D), jnp.float32),
        pltpu.VMEM_SHARED((ACC,), jnp.float32),
        pltpu.SemaphoreType.DMA,
        pltpu.SemaphoreType.DMA,
        pltpu.SemaphoreType.DMA,
        pltpu.SemaphoreType.DMA,
    ]
    n2 = n_chunks // 2
    assert n_chunks % 2 == 0

    def body(row_hbm, dst_hbm, hx_hbm, *rest):
        if with_cnt:
            out_hbm, cnt_hbm = rest[0], rest[1]
            rest = rest[2:]
        else:
            out_hbm = rest[0]
            cnt_hbm = None
            rest = rest[1:]
        (row2, dstA, dstB, msgA, msgB, ones_v, cz_v,
         acc_sh, cnt_sh, semA, semB, semDA, semDB) = rest

        c = lax.axis_index("c")
        s = lax.axis_index("s")
        wid = s * NC + c
        nv = jnp.clip((E - wid * epw) // CH, 0, n_chunks)

        pltpu.sync_copy(row_hbm.at[wid], row2)

        zer16 = jnp.zeros((16,), jnp.float32)

        def zrow(i, _):
            for j in range(D // 16):
                msgA[i, pl.ds(j * 16, 16)] = zer16
            return 0
        lax.fori_loop(0, CH, zrow, 0)

        for j in range(CH // 16):
            ones_v[pl.ds(j * 16, 16)] = jnp.ones((16,), jnp.float32)

        def zc(i, _):
            cz_v[pl.ds(i * 16, 16)] = zer16
            return 0
        lax.fori_loop(0, stripe // 16, zc, 0)

        off = 0
        while off < stripe:
            step_rows = min(CH, stripe - off)
            pltpu.sync_copy(msgA.at[pl.ds(0, step_rows), :],
                            acc_sh.at[pl.ds(s * stripe + off, step_rows), :])
            off += step_rows
        pltpu.sync_copy(cz_v, cnt_sh.at[pl.ds(s * stripe, stripe)])

        dbase = wid * epw
        pltpu.async_copy(dst_hbm.at[pl.ds(dbase, CH)], dstA, semDA)
        pltpu.async_copy(hx_hbm.at[row2.at[0]], msgA, semA)
        plsc.subcore_barrier()

        def step(j, _):
            a = 2 * j
            pltpu.async_copy(dst_hbm.at[pl.ds(dbase + (a + 1) * CH, CH)],
                             dstB, semDB)
            pltpu.async_copy(hx_hbm.at[row2.at[a + 1]], msgB, semB)
            pltpu.make_async_copy(hx_hbm.at[row2.at[a]], msgA, semA).wait()
            pltpu.make_async_copy(
                dst_hbm.at[pl.ds(dbase + a * CH, CH)], dstA, semDA).wait()

            @pl.when(a < nv)
            def _():
                pltpu.sync_copy(msgA, acc_sh.at[dstA], add=True)
                if with_cnt:
                    pltpu.sync_copy(ones_v, cnt_sh.at[dstA], add=True)

            @pl.when(j < n2 - 1)
            def _():
                pltpu.async_copy(dst_hbm.at[pl.ds(dbase + (a + 2) * CH, CH)],
                                 dstA, semDA)
                pltpu.async_copy(hx_hbm.at[row2.at[a + 2]], msgA, semA)

            pltpu.make_async_copy(hx_hbm.at[row2.at[a + 1]], msgB, semB).wait()
            pltpu.make_async_copy(
                dst_hbm.at[pl.ds(dbase + (a + 1) * CH, CH)], dstB, semDB).wait()

            @pl.when(a + 1 < nv)
            def _():
                pltpu.sync_copy(msgB, acc_sh.at[dstB], add=True)
                if with_cnt:
                    pltpu.sync_copy(ones_v, cnt_sh.at[dstB], add=True)
            return 0
        lax.fori_loop(0, n2, step, 0)
        plsc.subcore_barrier()

        pltpu.sync_copy(acc_sh.at[pl.ds(s * stripe, stripe), :],
                        out_hbm.at[c, pl.ds(s * stripe, stripe), :])
        if with_cnt:
            pltpu.sync_copy(cnt_sh.at[pl.ds(s * stripe, stripe)],
                            cnt_hbm.at[c, pl.ds(s * stripe, stripe)])

    return pl.kernel(body, mesh=mesh, out_type=out_type,
                     scratch_types=scratch)


def _make_prep(N, nrows):
    def prep_body(src_ref, et_ref, row_ref):
        row_ref[...] = et_ref[...] * N + src_ref[...]

    return pl.pallas_call(
        prep_body,
        out_shape=jax.ShapeDtypeStruct((nrows, CH), jnp.int32),
    )


def _expand_body(h_ref, bases_ref, comp_ref, root_ref, bias_ref,
                 hx_ref, rt_ref):
    h = h_ref[...]
    W = jnp.einsum('rb,bio->rio', comp_ref[...], bases_ref[...],
                   preferred_element_type=jnp.float32)
    hx_ref[...] = jnp.einsum('ni,rio->rno', h, W,
                             preferred_element_type=jnp.float32)
    rt_ref[...] = (jnp.dot(h, root_ref[...],
                           preferred_element_type=jnp.float32)
                   + bias_ref[...])


def _make_expand(N, D, R, B, nb):
    grid = (N // nb,)
    return pl.pallas_call(
        _expand_body,
        grid=grid,
        in_specs=[
            pl.BlockSpec((nb, D), lambda i: (i, 0)),
            pl.BlockSpec((B, D, D), lambda i: (0, 0, 0)),
            pl.BlockSpec((R, B), lambda i: (0, 0)),
            pl.BlockSpec((D, D), lambda i: (0, 0)),
            pl.BlockSpec((1, D), lambda i: (0, 0)),
        ],
        out_specs=[
            pl.BlockSpec((R, nb, D), lambda i: (0, i, 0)),
            pl.BlockSpec((nb, D), lambda i: (i, 0)),
        ],
        out_shape=[
            jax.ShapeDtypeStruct((R, N, D), jnp.float32),
            jax.ShapeDtypeStruct((N, D), jnp.float32),
        ],
    )


def _combine_body(relu, p_ref, cnt_ref, rt_ref, o_ref):
    sm = p_ref[0] + p_ref[1]
    cn = cnt_ref[0] + cnt_ref[1]
    o = sm / jnp.maximum(cn, 1.0) + rt_ref[...]
    if relu:
        o = jnp.maximum(o, 0.0)
    o_ref[...] = o


def _make_combine(N, D, ACC, nb, relu):
    grid = (N // nb,)
    return pl.pallas_call(
        functools.partial(_combine_body, relu),
        grid=grid,
        in_specs=[
            pl.BlockSpec((NC, nb, D), lambda i: (0, i, 0)),
            pl.BlockSpec((NC, nb, 1), lambda i: (0, i, 0)),
            pl.BlockSpec((nb, D), lambda i: (i, 0)),
        ],
        out_specs=pl.BlockSpec((nb, D), lambda i: (i, 0)),
        out_shape=jax.ShapeDtypeStruct((N, D), jnp.float32),
    )


def kernel(x, edge_index, edge_type,
           bases0, comp0, root0, bias0,
           bases1, comp1, root1, bias1,
           bases2, comp2, root2, bias2):
    N, D = x.shape
    E = edge_type.shape[0]
    R, B = comp0.shape

    epw = -(-E // (NW * 2 * CH)) * 2 * CH
    n_chunks = epw // CH
    EP = epw * NW
    padn = EP - E
    ACC = -(-N // (NS * 32)) * (NS * 32)

    src = edge_index[0]
    dst = edge_index[1]
    ar = jnp.arange(padn, dtype=jnp.int32)
    src_p = jnp.concatenate([src, ar % N])
    et_p = jnp.concatenate([edge_type, jnp.zeros((padn,), jnp.int32)])
    dst_p = jnp.concatenate([dst, jnp.zeros((padn,), jnp.int32)])

    prep = _make_prep(N, EP // CH)
    row_p = prep(src_p.reshape(EP // CH, CH),
                 et_p.reshape(EP // CH, CH)).reshape(NW, n_chunks, CH)

    expand = _make_expand(N, D, R, B, nb=1000)
    sc0 = _make_sc_scatter(N, D, ACC, E, epw, n_chunks, with_cnt=True)
    sc1 = _make_sc_scatter(N, D, ACC, E, epw, n_chunks, with_cnt=False)

    params = [(bases0, comp0, root0, bias0),
              (bases1, comp1, root1, bias1),
              (bases2, comp2, root2, bias2)]

    h = x
    cnt3 = None
    for li, (bases, comp, root, bias) in enumerate(params):
        hx, rt = expand(h, bases, comp, root, bias.reshape(1, D))
        hx_flat = hx.reshape(R * N, D)
        if li == 0:
            parts, cnt = sc0(row_p, dst_p, hx_flat)
            cnt3 = cnt.reshape(NC, ACC, 1)
        else:
            (parts,) = sc1(row_p, dst_p, hx_flat)
        combine = _make_combine(N, D, ACC, nb=1000, relu=(li < 2))
        h = combine(parts, cnt3, rt)
    return h


# speedup vs baseline: 40.5082x; 1.3084x over previous
import functools

import jax
import jax.numpy as jnp
from jax import lax
from jax.experimental import pallas as pl
from jax.experimental.pallas import tpu as pltpu
from jax.experimental.pallas import tpu_sc as plsc

NC = 2
NS = 16
NW = NC * NS
CH = 128


def _make_sc_scatter(N, D, ACC, E, epw, n_chunks, with_cnt):
    stripe = ACC // NS
    mesh = plsc.VectorSubcoreMesh(core_axis_name="c", subcore_axis_name="s")
    out_type = [jax.ShapeDtypeStruct((NC, ACC, D), jnp.float32)]
    if with_cnt:
        out_type.append(jax.ShapeDtypeStruct((NC, ACC), jnp.float32))
    scratch = [
        pltpu.VMEM((n_chunks, CH), jnp.int32),
        pltpu.VMEM((CH,), jnp.int32),
        pltpu.VMEM((CH,), jnp.int32),
        pltpu.VMEM((CH, D), jnp.float32),
        pltpu.VMEM((CH, D), jnp.float32),
        pltpu.VMEM((CH,), jnp.float32),
        pltpu.VMEM((stripe,), jnp.float32),
        pltpu.VMEM_SHARED((ACC, D), jnp.float32),
        pltpu.VMEM_SHARED((ACC,), jnp.float32),
        pltpu.SemaphoreType.DMA,
        pltpu.SemaphoreType.DMA,
        pltpu.SemaphoreType.DMA,
        pltpu.SemaphoreType.DMA,
    ]
    n2 = n_chunks // 2
    assert n_chunks % 2 == 0

    def body(row_hbm, dst_hbm, hx_hbm, *rest):
        if with_cnt:
            out_hbm, cnt_hbm = rest[0], rest[1]
            rest = rest[2:]
        else:
            out_hbm = rest[0]
            cnt_hbm = None
            rest = rest[1:]
        (row2, dstA, dstB, msgA, msgB, ones_v, cz_v,
         acc_sh, cnt_sh, semA, semB, semDA, semDB) = rest

        c = lax.axis_index("c")
        s = lax.axis_index("s")
        wid = s * NC + c
        nv = jnp.clip((E - wid * epw) // CH, 0, n_chunks)

        pltpu.sync_copy(row_hbm.at[wid], row2)

        zer16 = jnp.zeros((16,), jnp.float32)

        def zrow(i, _):
            for j in range(D // 16):
                msgA[i, pl.ds(j * 16, 16)] = zer16
            return 0
        lax.fori_loop(0, CH, zrow, 0)

        for j in range(CH // 16):
            ones_v[pl.ds(j * 16, 16)] = jnp.ones((16,), jnp.float32)

        def zc(i, _):
            cz_v[pl.ds(i * 16, 16)] = zer16
            return 0
        lax.fori_loop(0, stripe // 16, zc, 0)

        off = 0
        while off < stripe:
            step_rows = min(CH, stripe - off)
            pltpu.sync_copy(msgA.at[pl.ds(0, step_rows), :],
                            acc_sh.at[pl.ds(s * stripe + off, step_rows), :])
            off += step_rows
        pltpu.sync_copy(cz_v, cnt_sh.at[pl.ds(s * stripe, stripe)])

        dbase = wid * epw
        pltpu.async_copy(dst_hbm.at[pl.ds(dbase, CH)], dstA, semDA)
        pltpu.async_copy(hx_hbm.at[row2.at[0]], msgA, semA)
        plsc.subcore_barrier()

        def step(j, _):
            a = 2 * j
            pltpu.async_copy(dst_hbm.at[pl.ds(dbase + (a + 1) * CH, CH)],
                             dstB, semDB)
            pltpu.async_copy(hx_hbm.at[row2.at[a + 1]], msgB, semB)
            pltpu.make_async_copy(hx_hbm.at[row2.at[a]], msgA, semA).wait()
            pltpu.make_async_copy(
                dst_hbm.at[pl.ds(dbase + a * CH, CH)], dstA, semDA).wait()

            @pl.when(a < nv)
            def _():
                pltpu.sync_copy(msgA, acc_sh.at[dstA], add=True)
                if with_cnt:
                    pltpu.sync_copy(ones_v, cnt_sh.at[dstA], add=True)

            @pl.when(j < n2 - 1)
            def _():
                pltpu.async_copy(dst_hbm.at[pl.ds(dbase + (a + 2) * CH, CH)],
                                 dstA, semDA)
                pltpu.async_copy(hx_hbm.at[row2.at[a + 2]], msgA, semA)

            pltpu.make_async_copy(hx_hbm.at[row2.at[a + 1]], msgB, semB).wait()
            pltpu.make_async_copy(
                dst_hbm.at[pl.ds(dbase + (a + 1) * CH, CH)], dstB, semDB).wait()

            @pl.when(a + 1 < nv)
            def _():
                pltpu.sync_copy(msgB, acc_sh.at[dstB], add=True)
                if with_cnt:
                    pltpu.sync_copy(ones_v, cnt_sh.at[dstB], add=True)
            return 0
        lax.fori_loop(0, n2, step, 0)
        plsc.subcore_barrier()

        pltpu.sync_copy(acc_sh.at[pl.ds(s * stripe, stripe), :],
                        out_hbm.at[c, pl.ds(s * stripe, stripe), :])
        if with_cnt:
            pltpu.sync_copy(cnt_sh.at[pl.ds(s * stripe, stripe)],
                            cnt_hbm.at[c, pl.ds(s * stripe, stripe)])

    return pl.kernel(body, mesh=mesh, out_type=out_type,
                     scratch_types=scratch)


def _make_prep(N, nrows):
    def prep_body(src_ref, et_ref, row_ref):
        row_ref[...] = et_ref[...] * N + src_ref[...]

    return pl.pallas_call(
        prep_body,
        out_shape=jax.ShapeDtypeStruct((nrows, CH), jnp.int32),
    )


def _make_wprep(D, R, B, L):
    def wbody(comp_ref, bases_ref, w_ref):
        acc = None
        for b in range(B):
            term = comp_ref[0, :, b][:, None, None] * bases_ref[0, b][None]
            acc = term if acc is None else acc + term
        w_ref[0] = acc

    return pl.pallas_call(
        wbody,
        grid=(L,),
        in_specs=[
            pl.BlockSpec((1, R, B), lambda i: (i, 0, 0)),
            pl.BlockSpec((1, B, D, D), lambda i: (i, 0, 0, 0)),
        ],
        out_specs=pl.BlockSpec((1, R, D, D), lambda i: (i, 0, 0, 0)),
        out_shape=jax.ShapeDtypeStruct((L, R, D, D), jnp.float32),
    )


def _expand0_body(h_ref, w_ref, root_ref, bias_ref, hx_ref, rt_ref):
    h = h_ref[...]
    hx_ref[...] = jnp.einsum('ni,rio->rno', h, w_ref[...],
                             preferred_element_type=jnp.float32)
    rt_ref[...] = (jnp.dot(h, root_ref[...],
                           preferred_element_type=jnp.float32)
                   + bias_ref[...])


def _make_expand0(N, D, R, nb):
    return pl.pallas_call(
        _expand0_body,
        grid=(N // nb,),
        in_specs=[
            pl.BlockSpec((nb, D), lambda i: (i, 0)),
            pl.BlockSpec((R, D, D), lambda i: (0, 0, 0)),
            pl.BlockSpec((D, D), lambda i: (0, 0)),
            pl.BlockSpec((1, D), lambda i: (0, 0)),
        ],
        out_specs=[
            pl.BlockSpec((R, nb, D), lambda i: (0, i, 0)),
            pl.BlockSpec((nb, D), lambda i: (i, 0)),
        ],
        out_shape=[
            jax.ShapeDtypeStruct((R, N, D), jnp.float32),
            jax.ShapeDtypeStruct((N, D), jnp.float32),
        ],
    )


def _expandf_body(p_ref, cnt_ref, rtp_ref, w_ref, root_ref, bias_ref,
                  hx_ref, rt_ref):
    sm = p_ref[0] + p_ref[1]
    cn = cnt_ref[0] + cnt_ref[1]
    h = jnp.maximum(sm / jnp.maximum(cn, 1.0) + rtp_ref[...], 0.0)
    hx_ref[...] = jnp.einsum('ni,rio->rno', h, w_ref[...],
                             preferred_element_type=jnp.float32)
    rt_ref[...] = (jnp.dot(h, root_ref[...],
                           preferred_element_type=jnp.float32)
                   + bias_ref[...])


def _make_expandf(N, D, R, ACC, nb):
    return pl.pallas_call(
        _expandf_body,
        grid=(N // nb,),
        in_specs=[
            pl.BlockSpec((NC, nb, D), lambda i: (0, i, 0)),
            pl.BlockSpec((NC, nb, 1), lambda i: (0, i, 0)),
            pl.BlockSpec((nb, D), lambda i: (i, 0)),
            pl.BlockSpec((R, D, D), lambda i: (0, 0, 0)),
            pl.BlockSpec((D, D), lambda i: (0, 0)),
            pl.BlockSpec((1, D), lambda i: (0, 0)),
        ],
        out_specs=[
            pl.BlockSpec((R, nb, D), lambda i: (0, i, 0)),
            pl.BlockSpec((nb, D), lambda i: (i, 0)),
        ],
        out_shape=[
            jax.ShapeDtypeStruct((R, N, D), jnp.float32),
            jax.ShapeDtypeStruct((N, D), jnp.float32),
        ],
    )


def _combine_body(relu, p_ref, cnt_ref, rt_ref, o_ref):
    sm = p_ref[0] + p_ref[1]
    cn = cnt_ref[0] + cnt_ref[1]
    o = sm / jnp.maximum(cn, 1.0) + rt_ref[...]
    if relu:
        o = jnp.maximum(o, 0.0)
    o_ref[...] = o


def _make_combine(N, D, ACC, nb, relu):
    grid = (N // nb,)
    return pl.pallas_call(
        functools.partial(_combine_body, relu),
        grid=grid,
        in_specs=[
            pl.BlockSpec((NC, nb, D), lambda i: (0, i, 0)),
            pl.BlockSpec((NC, nb, 1), lambda i: (0, i, 0)),
            pl.BlockSpec((nb, D), lambda i: (i, 0)),
        ],
        out_specs=pl.BlockSpec((nb, D), lambda i: (i, 0)),
        out_shape=jax.ShapeDtypeStruct((N, D), jnp.float32),
    )


def kernel(x, edge_index, edge_type,
           bases0, comp0, root0, bias0,
           bases1, comp1, root1, bias1,
           bases2, comp2, root2, bias2):
    N, D = x.shape
    E = edge_type.shape[0]
    R, B = comp0.shape

    epw = -(-E // (NW * 2 * CH)) * 2 * CH
    n_chunks = epw // CH
    EP = epw * NW
    padn = EP - E
    ACC = -(-N // (NS * 32)) * (NS * 32)

    src = edge_index[0]
    dst = edge_index[1]
    ar = jnp.arange(padn, dtype=jnp.int32)
    src_p = jnp.concatenate([src, ar % N])
    et_p = jnp.concatenate([edge_type, jnp.zeros((padn,), jnp.int32)])
    dst_p = jnp.concatenate([dst, jnp.zeros((padn,), jnp.int32)])

    prep = _make_prep(N, EP // CH)
    row_p = prep(src_p.reshape(EP // CH, CH),
                 et_p.reshape(EP // CH, CH)).reshape(NW, n_chunks, CH)

    nb = 2000
    wprep = _make_wprep(D, R, B, 3)
    expand0 = _make_expand0(N, D, R, nb)
    expandf = _make_expandf(N, D, R, ACC, nb)
    sc0 = _make_sc_scatter(N, D, ACC, E, epw, n_chunks, with_cnt=True)
    sc1 = _make_sc_scatter(N, D, ACC, E, epw, n_chunks, with_cnt=False)

    comps = jnp.stack([comp0, comp1, comp2])
    basess = jnp.stack([bases0, bases1, bases2])
    Ws = wprep(comps, basess)

    roots = [root0, root1, root2]
    biases = [bias0, bias1, bias2]

    parts = cnt3 = rt = None
    for li in range(3):
        if li == 0:
            hx, rt = expand0(x, Ws[0], roots[0], biases[0].reshape(1, D))
        else:
            hx, rt = expandf(parts, cnt3, rt, Ws[li], roots[li],
                             biases[li].reshape(1, D))
        hx_flat = hx.reshape(R * N, D)
        if li == 0:
            parts, cnt = sc0(row_p, dst_p, hx_flat)
            cnt3 = cnt.reshape(NC, ACC, 1)
        else:
            (parts,) = sc1(row_p, dst_p, hx_flat)
    combine = _make_combine(N, D, ACC, nb=nb, relu=False)
    return combine(parts, cnt3, rt)
